# baseline (speedup 1.0000x reference)
import functools

import jax
import jax.numpy as jnp
from jax import lax
from jax.experimental import pallas as pl
from jax.experimental.pallas import tpu as pltpu
from jax.experimental.pallas import tpu_sc as plsc

D = 128
L = 20
NC = 2
NS = 16
NW = NC * NS
NBUF = 8
LOOKAHEAD = NBUF - 2


@functools.cache
def _make_gather(B):
    RPW = B // NW
    NCHUNK = RPW

    mesh = plsc.VectorSubcoreMesh(core_axis_name="c", subcore_axis_name="s")

    @functools.partial(
        pl.kernel,
        out_type=jax.ShapeDtypeStruct((B, L, D), jnp.float32),
        mesh=mesh,
        compiler_params=pltpu.CompilerParams(use_tc_tiling_on_sc=True),
        scratch_types=[
            pltpu.VMEM((RPW, L), jnp.int32),
            pltpu.VMEM((NBUF, L, D), jnp.float32),
            pltpu.SemaphoreType.DMA,
            pltpu.SemaphoreType.DMA,
        ],
    )
    def gather_kernel(idx_hbm, table_hbm, out_hbm, idx_v, rows_v, gsem, osem):
        wid = lax.axis_index("s") * NC + lax.axis_index("c")
        row0 = wid * RPW
        pltpu.sync_copy(idx_hbm.at[pl.ds(row0, RPW)], idx_v)

        def g_start(c):
            b = lax.rem(c, NBUF) if not isinstance(c, int) else c % NBUF
            pltpu.async_copy(
                table_hbm.at[idx_v.at[c]],
                rows_v.at[b],
                gsem,
            )

        def g_wait():
            pltpu.make_async_copy(out_hbm.at[row0], rows_v.at[0], gsem).wait()

        def o_start(c):
            b = lax.rem(c, NBUF) if not isinstance(c, int) else c % NBUF
            pltpu.async_copy(rows_v.at[b], out_hbm.at[row0 + c], osem)

        def o_wait():
            pltpu.make_async_copy(rows_v.at[0], out_hbm.at[row0], osem).wait()

        for c in range(LOOKAHEAD):
            g_start(c)

        @pl.loop(0, NCHUNK + 2)
        def body(c):
            @pl.when(c >= 2)
            def _():
                o_wait()

            @pl.when(c + LOOKAHEAD < NCHUNK)
            def _():
                g_start(c + LOOKAHEAD)

            @pl.when(c < NCHUNK)
            def _():
                g_wait()
                o_start(c)

    return gather_kernel


def kernel(input, weight):
    B, seq = input.shape
    out = _make_gather(B)(input.astype(jnp.int32), weight)
    return out


# speedup vs baseline: 1.5837x; 1.5837x over previous
import functools

import jax
import jax.numpy as jnp
from jax import lax
from jax.experimental import pallas as pl
from jax.experimental.pallas import tpu as pltpu
from jax.experimental.pallas import tpu_sc as plsc

D = 128
L = 20
NC = 2
NS = 16
NW = NC * NS
NBUF = 6
LOOKAHEAD = NBUF - 2
VL = 16


@functools.cache
def _make_gather(B):
    RPW = B // NW
    NCHUNK = L

    mesh = plsc.VectorSubcoreMesh(core_axis_name="c", subcore_axis_name="s")

    @functools.partial(
        pl.kernel,
        out_type=jax.ShapeDtypeStruct((L, B, D), jnp.float32),
        mesh=mesh,
        compiler_params=pltpu.CompilerParams(needs_layout_passes=False),
        scratch_types=[
            pltpu.VMEM((RPW, L), jnp.int32),
            pltpu.VMEM((L, RPW), jnp.int32),
            pltpu.VMEM((NBUF, RPW, D), jnp.float32),
            pltpu.SemaphoreType.DMA,
            pltpu.SemaphoreType.DMA,
        ],
    )
    def gather_kernel(
        idx_hbm, table_hbm, out_hbm, idx_v, idx_t, rows_v, gsem, osem
    ):
        wid = lax.axis_index("s") * NC + lax.axis_index("c")
        b0 = wid * RPW
        pltpu.sync_copy(idx_hbm.at[pl.ds(b0, RPW)], idx_v)

        lane = lax.iota(jnp.int32, VL)
        for l in range(L):
            col = jnp.full((VL,), l, jnp.int32)
            for j in range(RPW // VL):
                vec = plsc.load_gather(idx_v, [lane + (VL * j), col])
                idx_t[l, pl.ds(VL * j, VL)] = vec

        def g_start(c):
            b = lax.rem(c, NBUF) if not isinstance(c, int) else c % NBUF
            pltpu.async_copy(table_hbm.at[idx_t.at[c]], rows_v.at[b], gsem)

        def g_wait():
            pltpu.make_async_copy(
                out_hbm.at[0, pl.ds(b0, RPW), :], rows_v.at[0], gsem
            ).wait()

        def o_start(c):
            b = lax.rem(c, NBUF) if not isinstance(c, int) else c % NBUF
            pltpu.async_copy(
                rows_v.at[b], out_hbm.at[c, pl.ds(b0, RPW), :], osem
            )

        def o_wait():
            pltpu.make_async_copy(
                rows_v.at[0], out_hbm.at[0, pl.ds(b0, RPW), :], osem
            ).wait()

        for c in range(LOOKAHEAD):
            g_start(c)

        @pl.loop(0, NCHUNK + 2)
        def body(c):
            @pl.when(c >= 2)
            def _():
                o_wait()

            @pl.when(c + LOOKAHEAD < NCHUNK)
            def _():
                g_start(c + LOOKAHEAD)

            @pl.when(c < NCHUNK)
            def _():
                g_wait()
                o_start(c)

    return gather_kernel


def kernel(input, weight):
    B, seq = input.shape
    out = _make_gather(B)(input.astype(jnp.int32), weight)
    return jnp.transpose(out, (1, 0, 2))


# speedup vs baseline: 1.6310x; 1.0299x over previous
import functools

import jax
import jax.numpy as jnp
from jax import lax
from jax.experimental import pallas as pl
from jax.experimental.pallas import tpu as pltpu
from jax.experimental.pallas import tpu_sc as plsc

D = 128
L = 20
NC = 2
NS = 16
NW = NC * NS
NBUF = 6
LOOKAHEAD = NBUF - 2
VL = 16


@functools.cache
def _make_gather(B):
    RPW = B // NW
    NCHUNK = L

    mesh = plsc.VectorSubcoreMesh(core_axis_name="c", subcore_axis_name="s")

    @functools.partial(
        pl.kernel,
        out_type=jax.ShapeDtypeStruct((L, B, D), jnp.float32),
        mesh=mesh,
        compiler_params=pltpu.CompilerParams(needs_layout_passes=False),
        scratch_types=[
            pltpu.VMEM((RPW, L), jnp.int32),
            pltpu.VMEM((L, RPW), jnp.int32),
            pltpu.VMEM((NBUF, RPW, D), jnp.float32),
            pltpu.SemaphoreType.DMA,
            pltpu.SemaphoreType.DMA,
        ],
    )
    def gather_kernel(
        idx_hbm, table_hbm, out_hbm, idx_v, idx_t, rows_v, gsem, osem
    ):
        wid = lax.axis_index("s") * NC + lax.axis_index("c")
        b0 = wid * RPW
        pltpu.sync_copy(idx_hbm.at[pl.ds(b0, RPW)], idx_v)

        lane = lax.iota(jnp.int32, VL)

        def transpose_row(l):
            col = jnp.full((VL,), l, jnp.int32)
            for j in range(RPW // VL):
                vec = plsc.load_gather(idx_v, [lane + (VL * j), col])
                idx_t[l, pl.ds(VL * j, VL)] = vec

        def g_start(c):
            pltpu.async_copy(
                table_hbm.at[idx_t.at[c]], rows_v.at[c % NBUF], gsem
            )

        def g_wait():
            pltpu.make_async_copy(
                out_hbm.at[0, pl.ds(b0, RPW), :], rows_v.at[0], gsem
            ).wait()

        def o_start(c):
            pltpu.async_copy(
                rows_v.at[c % NBUF], out_hbm.at[c, pl.ds(b0, RPW), :], osem
            )

        def o_wait():
            pltpu.make_async_copy(
                rows_v.at[0], out_hbm.at[0, pl.ds(b0, RPW), :], osem
            ).wait()

        for c in range(LOOKAHEAD):
            transpose_row(c)
            g_start(c)
        for c in range(NCHUNK + 2):
            if 2 <= c:
                o_wait()
            if c + LOOKAHEAD < NCHUNK:
                transpose_row(c + LOOKAHEAD)
                g_start(c + LOOKAHEAD)
            if c < NCHUNK:
                g_wait()
                o_start(c)

    return gather_kernel


def kernel(input, weight):
    B, seq = input.shape
    out = _make_gather(B)(input.astype(jnp.int32), weight)
    return jnp.transpose(out, (1, 0, 2))


# speedup vs baseline: 1.6417x; 1.0065x over previous
import functools

import jax
import jax.numpy as jnp
from jax import lax
from jax.experimental import pallas as pl
from jax.experimental.pallas import tpu as pltpu
from jax.experimental.pallas import tpu_sc as plsc

D = 128
L = 20
NC = 2
NS = 16
NW = NC * NS
NBUF = 6
LOOKAHEAD = NBUF - 3
VL = 16


@functools.cache
def _make_gather(B):
    RPW = B // NW
    NCHUNK = L

    mesh = plsc.VectorSubcoreMesh(core_axis_name="c", subcore_axis_name="s")

    @functools.partial(
        pl.kernel,
        out_type=jax.ShapeDtypeStruct((L, B, D), jnp.float32),
        mesh=mesh,
        compiler_params=pltpu.CompilerParams(needs_layout_passes=False),
        scratch_types=[
            pltpu.VMEM((RPW, L), jnp.int32),
            pltpu.VMEM((L, RPW), jnp.int32),
            pltpu.VMEM((NBUF, RPW, D), jnp.float32),
            pltpu.SemaphoreType.DMA,
            pltpu.SemaphoreType.DMA,
        ],
    )
    def gather_kernel(
        idx_hbm, table_hbm, out_hbm, idx_v, idx_t, rows_v, gsem, osem
    ):
        wid = lax.axis_index("s") * NC + lax.axis_index("c")
        b0 = wid * RPW
        pltpu.sync_copy(idx_hbm.at[pl.ds(b0, RPW)], idx_v)

        lane = lax.iota(jnp.int32, VL)

        def transpose_row(l):
            col = jnp.full((VL,), l, jnp.int32)
            for j in range(RPW // VL):
                vec = plsc.load_gather(idx_v, [lane + (VL * j), col])
                idx_t[l, pl.ds(VL * j, VL)] = vec

        def g_start(c):
            pltpu.async_copy(
                table_hbm.at[idx_t.at[c]], rows_v.at[c % NBUF], gsem
            )

        def g_wait():
            pltpu.make_async_copy(
                out_hbm.at[0, pl.ds(b0, RPW), :], rows_v.at[0], gsem
            ).wait()

        def o_start(c):
            pltpu.async_copy(
                rows_v.at[c % NBUF], out_hbm.at[c, pl.ds(b0, RPW), :], osem
            )

        def o_wait():
            pltpu.make_async_copy(
                rows_v.at[0], out_hbm.at[0, pl.ds(b0, RPW), :], osem
            ).wait()

        for c in range(LOOKAHEAD):
            transpose_row(c)
            g_start(c)
        for c in range(NCHUNK + 2):
            if 2 <= c:
                o_wait()
            if c + LOOKAHEAD < NCHUNK:
                transpose_row(c + LOOKAHEAD)
                g_start(c + LOOKAHEAD)
            if c < NCHUNK:
                g_wait()
                o_start(c)

    return gather_kernel


def kernel(input, weight):
    B, seq = input.shape
    out = _make_gather(B)(input.astype(jnp.int32), weight)
    return jnp.transpose(out, (1, 0, 2))
